# baseline (speedup 1.0000x reference)
import functools

import jax
import jax.numpy as jnp
from jax.experimental import pallas as pl

N = 10000
D = 128
TM = 400


def _prelu(x, a):
    return jnp.where(x >= 0, x, a * x)


def _matmul_kernel(x_ref, w_ref, o_ref):
    o_ref[...] = jnp.dot(x_ref[...], w_ref[...],
                         preferred_element_type=jnp.float32)


def _layer1_kernel(adj_ref, y_ref, b_ref, a_ref, w2_ref, o_ref):
    h = jnp.dot(adj_ref[...], y_ref[...], preferred_element_type=jnp.float32)
    h = _prelu(h + b_ref[...], a_ref[0, 0])
    o_ref[...] = jnp.dot(h, w2_ref[...], preferred_element_type=jnp.float32)


def _layer2_kernel(adj_ref, y_ref, b_ref, a_ref, o_ref):
    h = jnp.dot(adj_ref[...], y_ref[...], preferred_element_type=jnp.float32)
    o_ref[...] = _prelu(h + b_ref[...], a_ref[0, 0])


@jax.jit
def _gcn(seq, adj, W1, b1, a1, W2, b2, a2):
    x = seq[0]
    b1r = b1.reshape(1, D)
    b2r = b2.reshape(1, D)
    a1r = a1.reshape(1, 1)
    a2r = a2.reshape(1, 1)

    y1 = pl.pallas_call(
        _matmul_kernel,
        out_shape=jax.ShapeDtypeStruct((N, D), jnp.float32),
    )(x, W1)

    grid = (N // TM,)
    adj_spec = pl.BlockSpec((TM, N), lambda m: (m, 0))
    feat_spec = pl.BlockSpec((N, D), lambda m: (0, 0))
    row_spec = pl.BlockSpec((1, D), lambda m: (0, 0))
    scalar_spec = pl.BlockSpec((1, 1), lambda m: (0, 0))
    out_spec = pl.BlockSpec((TM, D), lambda m: (m, 0))

    y2 = pl.pallas_call(
        _layer1_kernel,
        grid=grid,
        in_specs=[adj_spec, feat_spec, row_spec, scalar_spec,
                  pl.BlockSpec((D, D), lambda m: (0, 0))],
        out_specs=out_spec,
        out_shape=jax.ShapeDtypeStruct((N, D), jnp.float32),
    )(adj, y1, b1r, a1r, W2)

    h2 = pl.pallas_call(
        _layer2_kernel,
        grid=grid,
        in_specs=[adj_spec, feat_spec, row_spec, scalar_spec],
        out_specs=out_spec,
        out_shape=jax.ShapeDtypeStruct((N, D), jnp.float32),
    )(adj, y2, b2r, a2r)

    return h2[None, :, :]


def kernel(seq, adj, sparse, W1, b1, a1, W2, b2, a2):
    del sparse
    return _gcn(seq, adj, W1, b1, a1, W2, b2, a2)
